# baseline (speedup 1.0000x reference)
import functools

import jax
import jax.numpy as jnp
from jax import lax
from jax.experimental import pallas as pl
from jax.experimental.pallas import tpu as pltpu
from jax.experimental.pallas import tpu_sc as plsc

NUM_CLASSES = 19
ROW = 32
NBINS = 1024
L = 16
LANE_STRIDE = 1031
HSZ = L * LANE_STRIDE

NC = 2
NS = 16
NW = NC * NS

N_PIX = 16 * 512 * 512
PER_W = N_PIX // NW
CH_ROWS = 16
CH = CH_ROWS * 512
NCHUNK = PER_W // CH
NBUF = 4
VEC_PER_CH = CH // L
VEC_PER_ROW = 512 // L
UNROLL = 8
NSTREAM = 1

_mesh = plsc.VectorSubcoreMesh(core_axis_name="c", subcore_axis_name="s")


@functools.partial(
    pl.kernel,
    out_type=jax.ShapeDtypeStruct((NW * NBINS,), jnp.float32),
    mesh=_mesh,
    scratch_types=(
        [pltpu.VMEM((CH_ROWS, 512), jnp.int32) for _ in range(2 * NBUF)]
        + [
            pltpu.VMEM((NSTREAM * HSZ,), jnp.int32),
            pltpu.VMEM((NBINS,), jnp.float32),
        ]
        + [pltpu.SemaphoreType.DMA for _ in range(NBUF)]
    ),
    compiler_params=pltpu.CompilerParams(
        needs_layout_passes=False, use_tc_tiling_on_sc=True),
)
def _sc_hist(pred_hbm, label_hbm, out_hbm, *refs):
    bufs = [(refs[2 * b], refs[2 * b + 1], refs[2 * NBUF + 2 + b])
            for b in range(NBUF)]
    hist = refs[2 * NBUF]
    fhist = refs[2 * NBUF + 1]

    wid = lax.axis_index("s") * NC + lax.axis_index("c")
    img = wid // 2
    row0 = (wid % 2) * 256

    def _start(c):
        pb, lb, sm = bufs[c % NBUF]
        rows = pl.ds(row0 + c * CH_ROWS, CH_ROWS)
        cp = pltpu.async_copy(pred_hbm.at[img, rows, :], pb, sm)
        cl = pltpu.async_copy(label_hbm.at[img, rows, :], lb, sm)
        return cp, cl

    pending = [None] * NBUF
    for c in range(NBUF):
        pending[c] = _start(c)

    @plsc.parallel_loop(0, NSTREAM * HSZ // L, unroll=8)
    def _zero(i):
        hist[pl.ds(i * L, L)] = jnp.zeros((L,), jnp.int32)

    lane_base = lax.iota(jnp.int32, L) * LANE_STRIDE
    ones = jnp.ones((L,), jnp.int32)

    for c in range(NCHUNK):
        cp, cl = pending[c % NBUF]
        cp.wait()
        cl.wait()
        pb, lb, _ = bufs[c % NBUF]

        @plsc.parallel_loop(0, VEC_PER_CH, unroll=UNROLL)
        def _accum(i):
            r = i // VEC_PER_ROW
            coff = (i % VEC_PER_ROW) * L
            pv = pb[r, pl.ds(coff, L)]
            lv = lb[r, pl.ds(coff, L)]
            idx = lane_base + lv * ROW + pv
            plsc.addupdate_scatter(hist, [idx], ones)

        if c + NBUF < NCHUNK:
            pending[c % NBUF] = _start(c + NBUF)

    @plsc.parallel_loop(0, NBINS // L, unroll=2)
    def _reduce(b):
        acc = jnp.zeros((L,), jnp.int32)
        for s in range(NSTREAM):
            for lane in range(L):
                acc = acc + hist[
                    pl.ds(s * HSZ + lane * LANE_STRIDE + b * L, L)]
        fhist[pl.ds(b * L, L)] = acc.astype(jnp.float32)

    pltpu.sync_copy(fhist, out_hbm.at[pl.ds(wid * NBINS, NBINS)])


def _fold(part_ref, cm_ref, out_ref):
    s = part_ref[0:ROW, :]
    for w in range(1, NW):
        s = s + part_ref[w * ROW:(w + 1) * ROW, :]
    out_ref[...] = s[:NUM_CLASSES, :NUM_CLASSES] + cm_ref[...]


def kernel(imgPredict, imgLabel, confusionMatrix):
    partial = _sc_hist(imgPredict, imgLabel)
    part2d = partial.reshape(NW * ROW, ROW)
    return pl.pallas_call(
        _fold,
        out_shape=jax.ShapeDtypeStruct((NUM_CLASSES, NUM_CLASSES),
                                       jnp.float32),
    )(part2d, confusionMatrix)


# speedup vs baseline: 1.0235x; 1.0235x over previous
import functools

import jax
import jax.numpy as jnp
from jax import lax
from jax.experimental import pallas as pl
from jax.experimental.pallas import tpu as pltpu
from jax.experimental.pallas import tpu_sc as plsc

NUM_CLASSES = 19
ROW = 32
NBINS = 1024
L = 16
LANE_STRIDE = 1031
HSZ = L * LANE_STRIDE

NC = 2
NS = 16
NW = NC * NS

N_PIX = 16 * 512 * 512
PER_W = N_PIX // NW
CH_ROWS = 32
CH = CH_ROWS * 512
NCHUNK = PER_W // CH
NBUF = 2
VEC_PER_CH = CH // L
VEC_PER_ROW = 512 // L
UNROLL = 8
NSTREAM = 1

_mesh = plsc.VectorSubcoreMesh(core_axis_name="c", subcore_axis_name="s")


@functools.partial(
    pl.kernel,
    out_type=jax.ShapeDtypeStruct((NW * NBINS,), jnp.float32),
    mesh=_mesh,
    scratch_types=(
        [pltpu.VMEM((CH_ROWS, 512), jnp.int32) for _ in range(2 * NBUF)]
        + [
            pltpu.VMEM((NSTREAM * HSZ,), jnp.int32),
            pltpu.VMEM((NBINS,), jnp.float32),
        ]
        + [pltpu.SemaphoreType.DMA for _ in range(NBUF)]
    ),
    compiler_params=pltpu.CompilerParams(
        needs_layout_passes=False, use_tc_tiling_on_sc=True),
)
def _sc_hist(pred_hbm, label_hbm, out_hbm, *refs):
    bufs = [(refs[2 * b], refs[2 * b + 1], refs[2 * NBUF + 2 + b])
            for b in range(NBUF)]
    hist = refs[2 * NBUF]
    fhist = refs[2 * NBUF + 1]

    wid = lax.axis_index("s") * NC + lax.axis_index("c")
    img = wid // 2
    row0 = (wid % 2) * 256

    def _start(c):
        pb, lb, sm = bufs[c % NBUF]
        rows = pl.ds(row0 + c * CH_ROWS, CH_ROWS)
        cp = pltpu.async_copy(pred_hbm.at[img, rows, :], pb, sm)
        cl = pltpu.async_copy(label_hbm.at[img, rows, :], lb, sm)
        return cp, cl

    pending = [None] * NBUF
    for c in range(NBUF):
        pending[c] = _start(c)

    @plsc.parallel_loop(0, NSTREAM * HSZ // L, unroll=8)
    def _zero(i):
        hist[pl.ds(i * L, L)] = jnp.zeros((L,), jnp.int32)

    lane_base = lax.iota(jnp.int32, L) * LANE_STRIDE
    ones = jnp.ones((L,), jnp.int32)

    for c in range(NCHUNK):
        cp, cl = pending[c % NBUF]
        cp.wait()
        cl.wait()
        pb, lb, _ = bufs[c % NBUF]

        @plsc.parallel_loop(0, VEC_PER_CH, unroll=UNROLL)
        def _accum(i):
            r = i // VEC_PER_ROW
            coff = (i % VEC_PER_ROW) * L
            pv = pb[r, pl.ds(coff, L)]
            lv = lb[r, pl.ds(coff, L)]
            idx = lane_base + lv * ROW + pv
            plsc.addupdate_scatter(hist, [idx], ones)

        if c + NBUF < NCHUNK:
            pending[c % NBUF] = _start(c + NBUF)

    @plsc.parallel_loop(0, NBINS // L, unroll=2)
    def _reduce(b):
        acc = jnp.zeros((L,), jnp.int32)
        for s in range(NSTREAM):
            for lane in range(L):
                acc = acc + hist[
                    pl.ds(s * HSZ + lane * LANE_STRIDE + b * L, L)]
        fhist[pl.ds(b * L, L)] = acc.astype(jnp.float32)

    pltpu.sync_copy(fhist, out_hbm.at[pl.ds(wid * NBINS, NBINS)])


def _fold(part_ref, cm_ref, out_ref):
    s = part_ref[0:ROW, :]
    for w in range(1, NW):
        s = s + part_ref[w * ROW:(w + 1) * ROW, :]
    out_ref[...] = s[:NUM_CLASSES, :NUM_CLASSES] + cm_ref[...]


def kernel(imgPredict, imgLabel, confusionMatrix):
    partial = _sc_hist(imgPredict, imgLabel)
    part2d = partial.reshape(NW * ROW, ROW)
    return pl.pallas_call(
        _fold,
        out_shape=jax.ShapeDtypeStruct((NUM_CLASSES, NUM_CLASSES),
                                       jnp.float32),
    )(part2d, confusionMatrix)


# speedup vs baseline: 1.0740x; 1.0494x over previous
import functools

import jax
import jax.numpy as jnp
from jax import lax
from jax.experimental import pallas as pl
from jax.experimental.pallas import tpu as pltpu
from jax.experimental.pallas import tpu_sc as plsc

NUM_CLASSES = 19
ROW = 32
NBINS = 1024
L = 16
LANE_STRIDE = 1031
HSZ = L * LANE_STRIDE

NC = 2
NS = 16
NW = NC * NS

N_PIX = 16 * 512 * 512
PER_W = N_PIX // NW
CH_ROWS = 32
CH = CH_ROWS * 512
NCHUNK = PER_W // CH
NBUF = 2
VEC_PER_CH = CH // L
VEC_PER_ROW = 512 // L
UNROLL = 8
NSTREAM = 1

_mesh = plsc.VectorSubcoreMesh(core_axis_name="c", subcore_axis_name="s")


@functools.partial(
    pl.kernel,
    out_type=jax.ShapeDtypeStruct((NW, 8, 128), jnp.float32),
    mesh=_mesh,
    scratch_types=(
        [pltpu.VMEM((CH_ROWS, 512), jnp.int32) for _ in range(2 * NBUF)]
        + [
            pltpu.VMEM((NSTREAM * HSZ,), jnp.int32),
            pltpu.VMEM((8, 128), jnp.float32),
        ]
        + [pltpu.SemaphoreType.DMA for _ in range(NBUF)]
    ),
    compiler_params=pltpu.CompilerParams(
        needs_layout_passes=False, use_tc_tiling_on_sc=True),
)
def _sc_hist(pred_hbm, label_hbm, out_hbm, *refs):
    bufs = [(refs[2 * b], refs[2 * b + 1], refs[2 * NBUF + 2 + b])
            for b in range(NBUF)]
    hist = refs[2 * NBUF]
    fhist = refs[2 * NBUF + 1]

    wid = lax.axis_index("s") * NC + lax.axis_index("c")
    img = wid // 2
    row0 = (wid % 2) * 256

    def _start(c):
        pb, lb, sm = bufs[c % NBUF]
        rows = pl.ds(row0 + c * CH_ROWS, CH_ROWS)
        cp = pltpu.async_copy(pred_hbm.at[img, rows, :], pb, sm)
        cl = pltpu.async_copy(label_hbm.at[img, rows, :], lb, sm)
        return cp, cl

    pending = [None] * NBUF
    for c in range(NBUF):
        pending[c] = _start(c)

    @plsc.parallel_loop(0, NSTREAM * HSZ // L, unroll=8)
    def _zero(i):
        hist[pl.ds(i * L, L)] = jnp.zeros((L,), jnp.int32)

    lane_base = lax.iota(jnp.int32, L) * LANE_STRIDE
    ones = jnp.ones((L,), jnp.int32)

    for c in range(NCHUNK):
        cp, cl = pending[c % NBUF]
        cp.wait()
        cl.wait()
        pb, lb, _ = bufs[c % NBUF]

        @plsc.parallel_loop(0, VEC_PER_CH, unroll=UNROLL)
        def _accum(i):
            r = i // VEC_PER_ROW
            coff = (i % VEC_PER_ROW) * L
            pv = pb[r, pl.ds(coff, L)]
            lv = lb[r, pl.ds(coff, L)]
            idx = lane_base + lv * ROW + pv
            plsc.addupdate_scatter(hist, [idx], ones)

        if c + NBUF < NCHUNK:
            pending[c % NBUF] = _start(c + NBUF)

    @plsc.parallel_loop(0, NBINS // L, unroll=2)
    def _reduce(b):
        acc = jnp.zeros((L,), jnp.int32)
        for s in range(NSTREAM):
            for lane in range(L):
                acc = acc + hist[
                    pl.ds(s * HSZ + lane * LANE_STRIDE + b * L, L)]
        fhist[b // 8, pl.ds((b % 8) * L, L)] = acc.astype(jnp.float32)

    pltpu.sync_copy(fhist, out_hbm.at[wid])


def _fold(part_ref, cm_ref, out_ref):
    s = jnp.sum(part_ref[...], axis=0)
    rows = [s[l // 4:l // 4 + 1, (l % 4) * ROW:(l % 4) * ROW + NUM_CLASSES]
            for l in range(NUM_CLASSES)]
    out_ref[...] = jnp.concatenate(rows, axis=0) + cm_ref[...]


def kernel(imgPredict, imgLabel, confusionMatrix):
    partial = _sc_hist(imgPredict, imgLabel)
    return pl.pallas_call(
        _fold,
        out_shape=jax.ShapeDtypeStruct((NUM_CLASSES, NUM_CLASSES),
                                       jnp.float32),
    )(partial, confusionMatrix)
